# baseline (speedup 1.0000x reference)
import functools

import jax
import jax.numpy as jnp
from jax.experimental import pallas as pl
from jax.experimental.pallas import tpu as pltpu

HIDDEN = 2048
PROJ = 768
VOCAB = 100000
ROWS = 256
BV = 4096


def _proj_gelu_kernel(t_ref, w1_ref, b1_ref, x_ref):
    p = jax.lax.dot_general(
        t_ref[...], w1_ref[...], (((1,), (0,)), ((), ())),
        precision=jax.lax.Precision.HIGHEST,
        preferred_element_type=jnp.float32,
    ) + b1_ref[...]
    x_ref[...] = 0.5 * p * (1.0 + jax.lax.erf(p * 0.7071067811865476))


def _vocab_matmul_kernel(x_ref, w2_ref, b2_ref, out_ref):
    acc = jax.lax.dot_general(
        x_ref[...], w2_ref[...], (((1,), (0,)), ((), ())),
        precision=jax.lax.Precision.DEFAULT,
        preferred_element_type=jnp.float32,
    )
    out_ref[...] = acc + b2_ref[...]


@functools.partial(jax.jit, static_argnames=())
def kernel(t, W1, b1, W2, b2):
    B, S, _ = t.shape
    t2 = t.reshape(B * S, HIDDEN)
    x = pl.pallas_call(
        _proj_gelu_kernel,
        in_specs=[
            pl.BlockSpec((ROWS, HIDDEN), lambda: (0, 0)),
            pl.BlockSpec((HIDDEN, PROJ), lambda: (0, 0)),
            pl.BlockSpec((1, PROJ), lambda: (0, 0)),
        ],
        out_specs=pl.BlockSpec((ROWS, PROJ), lambda: (0, 0)),
        out_shape=jax.ShapeDtypeStruct((ROWS, PROJ), jnp.float32),
    )(t2, W1, b1.reshape(1, PROJ))

    nv = pl.cdiv(VOCAB, BV)
    out = pl.pallas_call(
        _vocab_matmul_kernel,
        grid=(nv,),
        in_specs=[
            pl.BlockSpec((ROWS, PROJ), lambda i: (0, 0)),
            pl.BlockSpec((PROJ, BV), lambda i: (0, i)),
            pl.BlockSpec((1, BV), lambda i: (0, i)),
        ],
        out_specs=pl.BlockSpec((ROWS, BV), lambda i: (0, i)),
        out_shape=jax.ShapeDtypeStruct((ROWS, VOCAB), jnp.float32),
        compiler_params=pltpu.CompilerParams(
            dimension_semantics=("parallel",),
        ),
    )(x, W2, b2.reshape(1, VOCAB))
    return out.reshape(B, S, VOCAB)
